# baseline (speedup 1.0000x reference)
import functools

import jax
import jax.numpy as jnp
from jax import lax
from jax.experimental import pallas as pl
from jax.experimental.pallas import tpu as pltpu
from jax.experimental.pallas import tpu_sc as plsc

VOCAB = 1000000
EMB = 64
FVOCAB = 100000
FEMB = 16
B = 4096
L = 50

NC = 2
NS = 16
NW = NC * NS
N = B * L
N_PER_W = N // NW
CHUNK = 128
NCHUNK = N_PER_W // CHUNK
OUT_D = EMB + 2 * FEMB

VBLK = 128
NFULL = (VOCAB // VBLK)
VTAIL = VOCAB - NFULL * VBLK


def _sc_format_word_table():
    mesh = plsc.VectorSubcoreMesh(core_axis_name="c", subcore_axis_name="s")

    @functools.partial(
        pl.kernel,
        out_type=jax.ShapeDtypeStruct((VOCAB * EMB,), jnp.float32),
        mesh=mesh,
        compiler_params=pltpu.CompilerParams(
            use_tc_tiling_on_sc=True, needs_layout_passes=False),
        scratch_types=[
            pltpu.VMEM((EMB, VBLK), jnp.float32),
            pltpu.VMEM((EMB, VBLK), jnp.float32),
            pltpu.VMEM((VBLK * EMB,), jnp.float32),
            pltpu.VMEM((VBLK * EMB,), jnp.float32),
            pltpu.SemaphoreType.DMA,
            pltpu.SemaphoreType.DMA,
            pltpu.SemaphoreType.DMA,
        ],
    )
    def k(wtT_hbm, tail_hbm, out_hbm, wblk0, wblk1, obuf0, obuf1,
          sem_i, sem_o0, sem_o1):
        wblks = (wblk0, wblk1)
        obufs = (obuf0, obuf1)
        wid = lax.axis_index("s") * NC + lax.axis_index("c")
        extra = jnp.where(wid < 4, 1, 0)
        nblk = 244 + extra
        blk0 = wid * 244 + jnp.minimum(wid, 4)

        lane = lax.iota(jnp.int32, 16)
        pre = [(lane + x0 * 16) * EMB for x0 in range(VBLK // 16)]

        @pl.when(wid == 0)
        def _():
            pltpu.sync_copy(tail_hbm, out_hbm.at[pl.ds(NFULL * VBLK * EMB,
                                                       VTAIL * EMB)])

        def load(b, s):
            pltpu.async_copy(
                wtT_hbm.at[:, pl.ds((blk0 + b) * VBLK, VBLK)], wblks[s],
                sem_i)

        def wait_load(b, s):
            pltpu.make_async_copy(
                wtT_hbm.at[:, pl.ds((blk0 + b) * VBLK, VBLK)], wblks[s],
                sem_i).wait()

        def store(b, s, sem):
            pltpu.async_copy(
                obufs[s], out_hbm.at[pl.ds((blk0 + b) * VBLK * EMB,
                                           VBLK * EMB)], sem)

        def wait_store(b, s, sem):
            pltpu.make_async_copy(
                obufs[s], out_hbm.at[pl.ds((blk0 + b) * VBLK * EMB,
                                           VBLK * EMB)], sem).wait()

        def transpose(s):
            os = obufs[s]
            for d in range(EMB):
                for x0 in range(VBLK // 16):
                    v = wblks[s][d, pl.ds(x0 * 16, 16)]
                    plsc.store_scatter(os, [pre[x0] + d], v)

        load(0, 0)

        def step(g, carry):
            b0 = g * 2
            pl.when(b0 + 1 < nblk)(lambda: load(b0 + 1, 1))
            wait_load(b0, 0)
            pl.when(g > 0)(lambda: wait_store(b0, 0, sem_o0))
            transpose(0)
            store(b0, 0, sem_o0)

            def odd():
                pl.when(b0 + 2 < nblk)(lambda: load(b0 + 2, 0))
                wait_load(b0 + 1, 1)
                pl.when(g > 0)(lambda: wait_store(b0 + 1, 1, sem_o1))
                transpose(1)
                store(b0 + 1, 1, sem_o1)

            pl.when(b0 + 1 < nblk)(odd)
            return carry

        lax.fori_loop(0, (nblk + 1) // 2, step, 0)
        wait_store(0, 0, sem_o0)
        pl.when(nblk > 1)(lambda: wait_store(0, 1, sem_o1))

    return k


def _sc_gather_concat():
    mesh = plsc.VectorSubcoreMesh(core_axis_name="c", subcore_axis_name="s")

    @functools.partial(
        pl.kernel,
        out_type=jax.ShapeDtypeStruct((N, OUT_D), jnp.float32),
        mesh=mesh,
        compiler_params=pltpu.CompilerParams(use_tc_tiling_on_sc=False),
        scratch_types=[
            pltpu.VMEM((NCHUNK, CHUNK), jnp.int32),
            pltpu.VMEM((NCHUNK, CHUNK), jnp.int32),
            pltpu.VMEM((NCHUNK, CHUNK), jnp.int32),
            pltpu.VMEM((2, CHUNK, EMB), jnp.float32),
            pltpu.VMEM((2, CHUNK, FEMB), jnp.float32),
            pltpu.VMEM((2, CHUNK, FEMB), jnp.float32),
            pltpu.SemaphoreType.DMA,
            pltpu.SemaphoreType.DMA,
            pltpu.SemaphoreType.DMA,
            pltpu.SemaphoreType.DMA,
            pltpu.SemaphoreType.DMA,
            pltpu.SemaphoreType.DMA,
        ],
    )
    def k(widx_hbm, f0idx_hbm, f1idx_hbm, wtab_hbm, f0tab_hbm, f1tab_hbm,
          out_hbm, widx_v, f0idx_v, f1idx_v, wrows, f0rows, f1rows,
          sem_gw, sem_g0, sem_g1, sem_ww, sem_w0, sem_w1):
        wid = lax.axis_index("s") * NC + lax.axis_index("c")
        base = wid * N_PER_W
        pltpu.sync_copy(widx_hbm.at[wid], widx_v)
        pltpu.sync_copy(f0idx_hbm.at[wid], f0idx_v)
        pltpu.sync_copy(f1idx_hbm.at[wid], f1idx_v)

        def gathers(j, s):
            pltpu.async_copy(wtab_hbm.at[widx_v.at[j]], wrows.at[s], sem_gw)
            pltpu.async_copy(f0tab_hbm.at[f0idx_v.at[j]], f0rows.at[s], sem_g0)
            pltpu.async_copy(f1tab_hbm.at[f1idx_v.at[j]], f1rows.at[s], sem_g1)

        def out_slices(j):
            row0 = base + j * CHUNK
            return (out_hbm.at[pl.ds(row0, CHUNK), pl.ds(0, EMB)],
                    out_hbm.at[pl.ds(row0, CHUNK), pl.ds(EMB, FEMB)],
                    out_hbm.at[pl.ds(row0, CHUNK), pl.ds(EMB + FEMB, FEMB)])

        def wait_writes(j, s):
            ow, o0, o1 = out_slices(j)
            pltpu.make_async_copy(wrows.at[s], ow, sem_ww).wait()
            pltpu.make_async_copy(f0rows.at[s], o0, sem_w0).wait()
            pltpu.make_async_copy(f1rows.at[s], o1, sem_w1).wait()

        gathers(0, 0)

        def step(j, carry):
            s = lax.rem(j, 2)
            pl.when(j >= 1)(lambda: wait_writes(j - 1, 1 - s))
            pl.when(j + 1 < NCHUNK)(lambda: gathers(j + 1, 1 - s))
            pltpu.make_async_copy(wtab_hbm.at[widx_v.at[j]], wrows.at[s], sem_gw).wait()
            pltpu.make_async_copy(f0tab_hbm.at[f0idx_v.at[j]], f0rows.at[s], sem_g0).wait()
            pltpu.make_async_copy(f1tab_hbm.at[f1idx_v.at[j]], f1rows.at[s], sem_g1).wait()
            ow, o0, o1 = out_slices(j)
            pltpu.async_copy(wrows.at[s], ow, sem_ww)
            pltpu.async_copy(f0rows.at[s], o0, sem_w0)
            pltpu.async_copy(f1rows.at[s], o1, sem_w1)
            return carry

        lax.fori_loop(0, NCHUNK, step, 0)
        wait_writes(NCHUNK - 1, (NCHUNK - 1) % 2)

    return k


_FORMAT_WORD = _sc_format_word_table()
_GATHER = _sc_gather_concat()


def kernel(word_inputs, feature_inputs_0, feature_inputs_1, word_seq_lengths,
           char_inputs, char_seq_lengths, char_seq_recover,
           word_table, feat_table_0, feat_table_1):
    widx = jnp.reshape(word_inputs.astype(jnp.int32), (NW, NCHUNK, CHUNK))
    f0idx = jnp.reshape(feature_inputs_0.astype(jnp.int32), (NW, NCHUNK, CHUNK))
    f1idx = jnp.reshape(feature_inputs_1.astype(jnp.int32), (NW, NCHUNK, CHUNK))
    tail = word_table[NFULL * VBLK:, :].reshape(-1)
    wt_flat = _FORMAT_WORD(word_table.T, tail)
    wt_lin = wt_flat.reshape(VOCAB, EMB)
    out = _GATHER(widx, f0idx, f1idx, wt_lin, feat_table_0, feat_table_1)
    return jnp.reshape(out, (B, L, OUT_D))


# speedup vs baseline: 1.6915x; 1.6915x over previous
import functools

import jax
import jax.numpy as jnp
from jax import lax
from jax.experimental import pallas as pl
from jax.experimental.pallas import tpu as pltpu
from jax.experimental.pallas import tpu_sc as plsc

VOCAB = 1000000
EMB = 64
FVOCAB = 100000
FEMB = 16
B = 4096
L = 50

NC = 2
NS = 16
NW = NC * NS
N = B * L
N_PER_W = N // NW
CHUNK = 128
NCHUNK = N_PER_W // CHUNK
OUT_D = EMB + 2 * FEMB

def _sc_gather_concat():
    mesh = plsc.VectorSubcoreMesh(core_axis_name="c", subcore_axis_name="s")

    @functools.partial(
        pl.kernel,
        out_type=jax.ShapeDtypeStruct((N, OUT_D), jnp.float32),
        mesh=mesh,
        compiler_params=pltpu.CompilerParams(use_tc_tiling_on_sc=False),
        scratch_types=[
            pltpu.VMEM((NCHUNK, CHUNK), jnp.int32),
            pltpu.VMEM((NCHUNK, CHUNK), jnp.int32),
            pltpu.VMEM((NCHUNK, CHUNK), jnp.int32),
            pltpu.VMEM((2, CHUNK, EMB), jnp.float32),
            pltpu.VMEM((2, CHUNK, FEMB), jnp.float32),
            pltpu.VMEM((2, CHUNK, FEMB), jnp.float32),
            pltpu.SemaphoreType.DMA,
            pltpu.SemaphoreType.DMA,
            pltpu.SemaphoreType.DMA,
            pltpu.SemaphoreType.DMA,
            pltpu.SemaphoreType.DMA,
            pltpu.SemaphoreType.DMA,
        ],
    )
    def k(widx_hbm, f0idx_hbm, f1idx_hbm, wtab_hbm, f0tab_hbm, f1tab_hbm,
          out_hbm, widx_v, f0idx_v, f1idx_v, wrows, f0rows, f1rows,
          sem_gw, sem_g0, sem_g1, sem_ww, sem_w0, sem_w1):
        wid = lax.axis_index("s") * NC + lax.axis_index("c")
        base = wid * N_PER_W
        pltpu.sync_copy(widx_hbm.at[wid], widx_v)
        pltpu.sync_copy(f0idx_hbm.at[wid], f0idx_v)
        pltpu.sync_copy(f1idx_hbm.at[wid], f1idx_v)

        def gathers(j, s):
            pltpu.async_copy(wtab_hbm.at[widx_v.at[j]], wrows.at[s], sem_gw)
            pltpu.async_copy(f0tab_hbm.at[f0idx_v.at[j]], f0rows.at[s], sem_g0)
            pltpu.async_copy(f1tab_hbm.at[f1idx_v.at[j]], f1rows.at[s], sem_g1)

        def out_slices(j):
            row0 = base + j * CHUNK
            return (out_hbm.at[pl.ds(row0, CHUNK), pl.ds(0, EMB)],
                    out_hbm.at[pl.ds(row0, CHUNK), pl.ds(EMB, FEMB)],
                    out_hbm.at[pl.ds(row0, CHUNK), pl.ds(EMB + FEMB, FEMB)])

        def wait_writes(j, s):
            ow, o0, o1 = out_slices(j)
            pltpu.make_async_copy(wrows.at[s], ow, sem_ww).wait()
            pltpu.make_async_copy(f0rows.at[s], o0, sem_w0).wait()
            pltpu.make_async_copy(f1rows.at[s], o1, sem_w1).wait()

        gathers(0, 0)

        def step(j, carry):
            s = lax.rem(j, 2)
            pl.when(j >= 1)(lambda: wait_writes(j - 1, 1 - s))
            pl.when(j + 1 < NCHUNK)(lambda: gathers(j + 1, 1 - s))
            pltpu.make_async_copy(wtab_hbm.at[widx_v.at[j]], wrows.at[s], sem_gw).wait()
            pltpu.make_async_copy(f0tab_hbm.at[f0idx_v.at[j]], f0rows.at[s], sem_g0).wait()
            pltpu.make_async_copy(f1tab_hbm.at[f1idx_v.at[j]], f1rows.at[s], sem_g1).wait()
            ow, o0, o1 = out_slices(j)
            pltpu.async_copy(wrows.at[s], ow, sem_ww)
            pltpu.async_copy(f0rows.at[s], o0, sem_w0)
            pltpu.async_copy(f1rows.at[s], o1, sem_w1)
            return carry

        lax.fori_loop(0, NCHUNK, step, 0)
        wait_writes(NCHUNK - 1, (NCHUNK - 1) % 2)

    return k


_GATHER = _sc_gather_concat()


def kernel(word_inputs, feature_inputs_0, feature_inputs_1, word_seq_lengths,
           char_inputs, char_seq_lengths, char_seq_recover,
           word_table, feat_table_0, feat_table_1):
    widx = jnp.reshape(word_inputs.astype(jnp.int32), (NW, NCHUNK, CHUNK))
    f0idx = jnp.reshape(feature_inputs_0.astype(jnp.int32), (NW, NCHUNK, CHUNK))
    f1idx = jnp.reshape(feature_inputs_1.astype(jnp.int32), (NW, NCHUNK, CHUNK))
    out = _GATHER(widx, f0idx, f1idx, word_table, feat_table_0, feat_table_1)
    return jnp.reshape(out, (B, L, OUT_D))
